# baseline (speedup 1.0000x reference)
import functools

import jax
import jax.numpy as jnp
from jax import lax
from jax.experimental import pallas as pl
from jax.experimental.pallas import tpu as pltpu
from jax.experimental.pallas import tpu_sc as plsc

B = 16
N = 8192
NSAMP = 2048
L = 16
NCHUNK = N // L


def _fps_body(xyz_hbm, out_hbm, x_v, y_v, z_v, dist_v, idx_v):
    nc = lax.axis_size("c")
    b = lax.axis_index("s") * nc + lax.axis_index("c")

    @pl.when(b < B)
    def _():
        pltpu.sync_copy(xyz_hbm.at[pl.ds(b * 3 * N, N)], x_v)
        pltpu.sync_copy(xyz_hbm.at[pl.ds((b * 3 + 1) * N, N)], y_v)
        pltpu.sync_copy(xyz_hbm.at[pl.ds((b * 3 + 2) * N, N)], z_v)

        def init_chunk(j, carry):
            dist_v[pl.ds(j * L, L)] = jnp.full((L,), 1e10, jnp.float32)
            return carry

        lax.fori_loop(0, NCHUNK, init_chunk, 0)

        lanes = lax.iota(jnp.int32, L)

        def fps_round(i, far):
            fvec = jnp.full((L,), far, jnp.int32)
            cxv = plsc.load_gather(x_v, [fvec])
            cyv = plsc.load_gather(y_v, [fvec])
            czv = plsc.load_gather(z_v, [fvec])
            plsc.store_scatter(
                idx_v, [jnp.full((L,), i, jnp.int32)], fvec, mask=lanes == 0
            )

            def chunk(j, carry):
                rmax, ridx = carry
                sl = pl.ds(j * L, L)
                dx = x_v[sl] - cxv
                dy = y_v[sl] - cyv
                dz = z_v[sl] - czv
                d = dx * dx + dy * dy + dz * dz
                nd = jnp.minimum(dist_v[sl], d)
                dist_v[sl] = nd
                m = nd > rmax
                rmax = jnp.where(m, nd, rmax)
                ridx = jnp.where(m, jnp.full((L,), j, jnp.int32), ridx)
                return rmax, ridx

            rmax, ridx = lax.fori_loop(
                0, NCHUNK, chunk,
                (jnp.full((L,), -1.0, jnp.float32), jnp.zeros((L,), jnp.int32)),
                unroll=8,
            )
            gmax = jnp.max(rmax)
            gidx = ridx * L + lanes
            cand = jnp.where(rmax == gmax, gidx, jnp.int32(2**30))
            return jnp.min(cand)

        lax.fori_loop(0, NSAMP, fps_round, jnp.int32(0))
        pltpu.sync_copy(idx_v, out_hbm.at[pl.ds(b * NSAMP, NSAMP)])


@jax.jit
def _fps(xyz):
    mesh = plsc.VectorSubcoreMesh(core_axis_name="c", subcore_axis_name="s")
    flat = pl.kernel(
        _fps_body,
        out_type=jax.ShapeDtypeStruct((B * NSAMP,), jnp.int32),
        mesh=mesh,
        compiler_params=pltpu.CompilerParams(needs_layout_passes=False),
        scratch_types=[
            pltpu.VMEM((N,), jnp.float32),
            pltpu.VMEM((N,), jnp.float32),
            pltpu.VMEM((N,), jnp.float32),
            pltpu.VMEM((N,), jnp.float32),
            pltpu.VMEM((NSAMP,), jnp.int32),
        ],
    )(xyz.reshape(B * 3 * N))
    return flat.reshape(B, NSAMP)


def kernel(xyz):
    return _fps(xyz)


# speedup vs baseline: 3.7983x; 3.7983x over previous
import functools

import jax
import jax.numpy as jnp
from jax import lax
from jax.experimental import pallas as pl
from jax.experimental.pallas import tpu as pltpu
from jax.experimental.pallas import tpu_sc as plsc

B = 16
N = 8192
NSAMP = 2048
L = 16
NCHUNK = N // L


def _fps_body(xyz_hbm, out_hbm, x_v, y_v, z_v, dist_v, idx_v):
    nc = lax.axis_size("c")
    b = lax.axis_index("s") * nc + lax.axis_index("c")

    @pl.when(b < B)
    def _():
        pltpu.sync_copy(xyz_hbm.at[pl.ds(b * 3 * N, N)], x_v)
        pltpu.sync_copy(xyz_hbm.at[pl.ds((b * 3 + 1) * N, N)], y_v)
        pltpu.sync_copy(xyz_hbm.at[pl.ds((b * 3 + 2) * N, N)], z_v)

        def init_chunk(j, carry):
            dist_v[pl.ds(j * L, L)] = jnp.full((L,), 1e10, jnp.float32)
            return carry

        lax.fori_loop(0, NCHUNK, init_chunk, 0)

        lanes = lax.iota(jnp.int32, L)

        def fps_round(i, far):
            fvec = jnp.full((L,), far, jnp.int32)
            cxv = plsc.load_gather(x_v, [fvec])
            cyv = plsc.load_gather(y_v, [fvec])
            czv = plsc.load_gather(z_v, [fvec])
            plsc.store_scatter(
                idx_v, [jnp.full((L,), i, jnp.int32)], fvec, mask=lanes == 0
            )

            carry0 = (jnp.full((L,), -1.0, jnp.float32),
                      jnp.zeros((L,), jnp.int32))

            @plsc.parallel_loop(0, NCHUNK, step=1, unroll=8, carry=carry0)
            def chunk(j, carry):
                rmax, ridx = carry
                sl = pl.ds(j * L, L)
                dx = x_v[sl] - cxv
                dy = y_v[sl] - cyv
                dz = z_v[sl] - czv
                d = dx * dx + dy * dy + dz * dz
                nd = jnp.minimum(dist_v[sl], d)
                dist_v[sl] = nd
                m = nd > rmax
                rmax = jnp.where(m, nd, rmax)
                ridx = jnp.where(m, jnp.full((L,), j, jnp.int32), ridx)
                return rmax, ridx

            rmax, ridx = chunk
            gmax = jnp.max(rmax)
            gidx = ridx * L + lanes
            cand = jnp.where(rmax == gmax, gidx, jnp.int32(2**30))
            return jnp.min(cand)

        lax.fori_loop(0, NSAMP, fps_round, jnp.int32(0))
        pltpu.sync_copy(idx_v, out_hbm.at[pl.ds(b * NSAMP, NSAMP)])


@jax.jit
def _fps(xyz):
    mesh = plsc.VectorSubcoreMesh(core_axis_name="c", subcore_axis_name="s")
    flat = pl.kernel(
        _fps_body,
        out_type=jax.ShapeDtypeStruct((B * NSAMP,), jnp.int32),
        mesh=mesh,
        compiler_params=pltpu.CompilerParams(needs_layout_passes=False),
        scratch_types=[
            pltpu.VMEM((N,), jnp.float32),
            pltpu.VMEM((N,), jnp.float32),
            pltpu.VMEM((N,), jnp.float32),
            pltpu.VMEM((N,), jnp.float32),
            pltpu.VMEM((NSAMP,), jnp.int32),
        ],
    )(xyz.reshape(B * 3 * N))
    return flat.reshape(B, NSAMP)


def kernel(xyz):
    return _fps(xyz)
